# baseline (speedup 1.0000x reference)
import jax
import jax.numpy as jnp
from jax.experimental import pallas as pl
from jax.experimental.pallas import tpu as pltpu

_NUM_EXPERTS = 8
_RANK = 64
_TILE = 128


def _moe_body(x_ref, wg_ref, a_ref, m_ref, out_ref, imp_ref, load_ref, a_sc):
    @pl.when(pl.program_id(0) == 0)
    def _prep():
        imp_ref[...] = jnp.zeros_like(imp_ref)
        load_ref[...] = jnp.zeros_like(load_ref)
        a_sc[...] = a_ref[...].astype(jnp.bfloat16)

    x = x_ref[...]
    logits = jnp.dot(x, wg_ref[...], preferred_element_type=jnp.float32)
    iota_e = jax.lax.broadcasted_iota(jnp.int32, logits.shape, 1)
    mx = jnp.max(logits, axis=1, keepdims=True)
    eid = jnp.min(
        jnp.where(logits >= mx, iota_e, _NUM_EXPERTS), axis=1, keepdims=True
    )

    counts = jnp.sum((iota_e == eid).astype(jnp.float32), axis=0)
    imp_ref[...] += counts[None, :]
    load_ref[...] += counts[None, :].astype(jnp.int32)

    xb = x.astype(jnp.bfloat16)
    h = jax.lax.dot_general(
        xb, a_sc[...], (((1,), (1,)), ((), ())),
        preferred_element_type=jnp.float32)
    col_e = jax.lax.broadcasted_iota(jnp.int32, h.shape, 1) // _RANK
    hg = jnp.where(col_e == eid, h, 0.0).astype(jnp.bfloat16)

    out_ref[...] = jnp.dot(
        hg, m_ref[...], preferred_element_type=jnp.float32)


def kernel(tokens, w_gate, A, B):
    b, s, d = tokens.shape
    e, r, _ = A.shape
    d_out = B.shape[1]
    flat = tokens.reshape(s, d)
    a2 = A.reshape(e * r, d)
    m = jnp.transpose(B, (0, 2, 1)).reshape(e * r, d_out).astype(jnp.bfloat16)

    n_tiles = s // _TILE
    out, imp, load = pl.pallas_call(
        _moe_body,
        grid=(n_tiles,),
        in_specs=[
            pl.BlockSpec((_TILE, d), lambda i: (i, 0)),
            pl.BlockSpec((d, e), lambda i: (0, 0)),
            pl.BlockSpec((e * r, d), lambda i: (0, 0)),
            pl.BlockSpec((e * r, d_out), lambda i: (0, 0)),
        ],
        out_specs=[
            pl.BlockSpec((_TILE, d_out), lambda i: (i, 0)),
            pl.BlockSpec((1, e), lambda i: (0, 0)),
            pl.BlockSpec((1, e), lambda i: (0, 0)),
        ],
        out_shape=[
            jax.ShapeDtypeStruct((s, d_out), jnp.float32),
            jax.ShapeDtypeStruct((1, e), jnp.float32),
            jax.ShapeDtypeStruct((1, e), jnp.int32),
        ],
        scratch_shapes=[
            pltpu.VMEM((e * r, d), jnp.bfloat16),
        ],
    )(flat, w_gate, a2, m)
    return out.reshape(b, s, d_out), imp.reshape(e), load.reshape(e)


# speedup vs baseline: 1.1238x; 1.1238x over previous
import jax
import jax.numpy as jnp
from jax.experimental import pallas as pl
from jax.experimental.pallas import tpu as pltpu

_NUM_EXPERTS = 8
_RANK = 64
_TILE = 256


def _moe_body(x_ref, wg_ref, a_ref, m_ref, out_ref, imp_ref, load_ref, a_sc):
    @pl.when(pl.program_id(0) == 0)
    def _prep():
        imp_ref[...] = jnp.zeros_like(imp_ref)
        load_ref[...] = jnp.zeros_like(load_ref)
        a_sc[...] = a_ref[...].astype(jnp.bfloat16)

    x = x_ref[...]
    logits = jnp.dot(x, wg_ref[...], preferred_element_type=jnp.float32)
    iota_e = jax.lax.broadcasted_iota(jnp.int32, logits.shape, 1)
    mx = jnp.max(logits, axis=1, keepdims=True)
    eid = jnp.min(
        jnp.where(logits >= mx, iota_e, _NUM_EXPERTS), axis=1, keepdims=True
    )

    counts = jnp.sum((iota_e == eid).astype(jnp.float32), axis=0)
    imp_ref[...] += counts[None, :]
    load_ref[...] += counts[None, :].astype(jnp.int32)

    xb = x.astype(jnp.bfloat16)
    h = jax.lax.dot_general(
        xb, a_sc[...], (((1,), (1,)), ((), ())),
        preferred_element_type=jnp.float32)
    col_e = jax.lax.broadcasted_iota(jnp.int32, h.shape, 1) // _RANK
    hg = jnp.where(col_e == eid, h, 0.0).astype(jnp.bfloat16)

    out_ref[...] = jnp.dot(
        hg, m_ref[...], preferred_element_type=jnp.float32)


def kernel(tokens, w_gate, A, B):
    b, s, d = tokens.shape
    e, r, _ = A.shape
    d_out = B.shape[1]
    flat = tokens.reshape(s, d)
    a2 = A.reshape(e * r, d)
    m = jnp.transpose(B, (0, 2, 1)).reshape(e * r, d_out).astype(jnp.bfloat16)

    n_tiles = s // _TILE
    out, imp, load = pl.pallas_call(
        _moe_body,
        grid=(n_tiles,),
        in_specs=[
            pl.BlockSpec((_TILE, d), lambda i: (i, 0)),
            pl.BlockSpec((d, e), lambda i: (0, 0)),
            pl.BlockSpec((e * r, d), lambda i: (0, 0)),
            pl.BlockSpec((e * r, d_out), lambda i: (0, 0)),
        ],
        out_specs=[
            pl.BlockSpec((_TILE, d_out), lambda i: (i, 0)),
            pl.BlockSpec((1, e), lambda i: (0, 0)),
            pl.BlockSpec((1, e), lambda i: (0, 0)),
        ],
        out_shape=[
            jax.ShapeDtypeStruct((s, d_out), jnp.float32),
            jax.ShapeDtypeStruct((1, e), jnp.float32),
            jax.ShapeDtypeStruct((1, e), jnp.int32),
        ],
        scratch_shapes=[
            pltpu.VMEM((e * r, d), jnp.bfloat16),
        ],
    )(flat, w_gate, a2, m)
    return out.reshape(b, s, d_out), imp.reshape(e), load.reshape(e)
